# Initial kernel scaffold; baseline (speedup 1.0000x reference)
import jax
import jax.numpy as jnp
from jax.experimental import pallas as pl


def kernel(x, edge_index, edge_attr, batch, node_ids, emb_table, W1, b1, W2, b2, Wfc, bfc):
    raise NotImplementedError("write your pallas kernel here")


# speedup vs baseline: 9.9894x; 9.9894x over previous
import functools

import jax
import jax.numpy as jnp
from jax import lax
from jax.experimental import pallas as pl
from jax.experimental.pallas import tpu as pltpu
from jax.experimental.pallas import tpu_sc as plsc

N = 10000
E = 320000
D = 128
ED = 128
H = 128
OUT = 32
G = 64

NC = 2
NS = 16
NW = NC * NS

K = 128
CH = 79
EPT = CH * K
EP = EPT * NW
RPT = 320
NP = RPT * NW
RPS = NP // NS

_mesh = plsc.VectorSubcoreMesh(core_axis_name="c", subcore_axis_name="s")


@functools.partial(
    pl.kernel,
    out_type=(
        jax.ShapeDtypeStruct((NC * NP,), jnp.float32),
        jax.ShapeDtypeStruct((NP, ED), jnp.float32),
    ),
    mesh=_mesh,
    scratch_types=[
        pltpu.VMEM((CH, K), jnp.int32),
        pltpu.VMEM((CH, K), jnp.float32),
        pltpu.VMEM((RPT,), jnp.int32),
        pltpu.VMEM((K, ED), jnp.float32),
        pltpu.VMEM((RPT,), jnp.float32),
        pltpu.VMEM_SHARED((NP,), jnp.float32),
        pltpu.SemaphoreType.DMA,
    ],
)
def _sc_deg_embed(dst_h, w_h, ids_h, emb_h, deg_out, emb_out,
                  dst_v, w_v, ids_v, rows_v, zrow_v, deg_sh, sem):
    cid = lax.axis_index("c")
    sid = lax.axis_index("s")
    wid = cid * NS + sid

    def z16(i, carry):
        zrow_v[pl.ds(i * 16, 16)] = jnp.zeros((16,), jnp.float32)
        return carry

    lax.fori_loop(0, RPT // 16, z16, 0)
    for i in range(NP // NS // RPT):
        pltpu.sync_copy(zrow_v, deg_sh.at[pl.ds((sid * (NP // NS // RPT) + i) * RPT, RPT)])
    pltpu.sync_copy(dst_h.at[wid], dst_v)
    pltpu.sync_copy(w_h.at[wid], w_v)
    plsc.subcore_barrier()

    def chunk(c, carry):
        pltpu.sync_copy(w_v.at[c], deg_sh.at[dst_v.at[c]], add=True)
        return carry

    lax.fori_loop(0, CH, chunk, 0)
    plsc.subcore_barrier()
    for i in range(NP // NS // RPT):
        off = (sid * (NP // NS // RPT) + i) * RPT
        pltpu.sync_copy(deg_sh.at[pl.ds(off, RPT)], zrow_v)
        pltpu.sync_copy(zrow_v, deg_out.at[pl.ds(cid * NP + off, RPT)])

    pltpu.sync_copy(ids_h.at[pl.ds(wid * RPT, RPT)], ids_v)
    for off, sz in ((0, 128), (128, 128), (256, 64)):
        pltpu.async_copy(emb_h.at[ids_v.at[pl.ds(off, sz)]],
                         rows_v.at[pl.ds(0, sz)], sem).wait()
        pltpu.sync_copy(rows_v.at[pl.ds(0, sz)],
                        emb_out.at[pl.ds(wid * RPT + off, sz)])


@functools.partial(
    pl.kernel,
    out_type=jax.ShapeDtypeStruct((NC, NP, H), jnp.float32),
    mesh=_mesh,
    scratch_types=[
        pltpu.VMEM((CH, K), jnp.int32),
        pltpu.VMEM((CH, K), jnp.int32),
        pltpu.VMEM((CH, K), jnp.float32),
        pltpu.VMEM((K, H), jnp.float32),
        pltpu.VMEM_SHARED((NP, H), jnp.float32),
        pltpu.SemaphoreType.DMA,
    ],
)
def _sc_edge_agg(src_h, dst_h, w_h, hts_h, agg_out,
                 src_v, dst_v, w_v, rows_v, acc_sh, sem):
    cid = lax.axis_index("c")
    sid = lax.axis_index("s")
    wid = cid * NS + sid

    def zrow(r, carry):
        for k8 in range(H // 16):
            rows_v[r, pl.ds(k8 * 16, 16)] = jnp.zeros((16,), jnp.float32)
        return carry

    lax.fori_loop(0, K, zrow, 0)
    for i in range(RPS // K):
        pltpu.sync_copy(rows_v, acc_sh.at[pl.ds(sid * RPS + i * K, K)])
    pltpu.sync_copy(src_h.at[wid], src_v)
    pltpu.sync_copy(dst_h.at[wid], dst_v)
    pltpu.sync_copy(w_h.at[wid], w_v)
    plsc.subcore_barrier()

    def chunk(c, carry):
        pltpu.async_copy(hts_h.at[src_v.at[c]], rows_v, sem).wait()

        def scale(g, carry2):
            wv = w_v[c, pl.ds(g * 16, 16)]
            base = g * 16
            for j in range(16):
                s = wv[j]
                for k8 in range(H // 16):
                    rows_v[base + j, pl.ds(k8 * 16, 16)] = (
                        rows_v[base + j, pl.ds(k8 * 16, 16)] * s)
            return carry2

        lax.fori_loop(0, K // 16, scale, 0)
        pltpu.sync_copy(rows_v, acc_sh.at[dst_v.at[c]], add=True)
        return carry

    lax.fori_loop(0, CH, chunk, 0)
    plsc.subcore_barrier()
    for i in range(RPS // K):
        pltpu.sync_copy(acc_sh.at[pl.ds(sid * RPS + i * K, K)], rows_v)
        pltpu.sync_copy(rows_v, agg_out.at[cid, pl.ds(sid * RPS + i * K, K)])


def _tc1_body(xp_ref, emb_ref, w1a_ref, w1b_ref, degp_ref, hts_ref, dinv_ref):
    deg = degp_ref[0] + degp_ref[1] + 1.0
    dinv = jnp.where(deg > 0, lax.rsqrt(deg), 0.0)
    ht = (jnp.dot(xp_ref[...], w1a_ref[...], preferred_element_type=jnp.float32)
          + jnp.dot(emb_ref[...], w1b_ref[...], preferred_element_type=jnp.float32))
    hts_ref[...] = ht * dinv
    dinv_ref[...] = dinv


def _tc2_body(agg_ref, hts_ref, dinv_ref, b_ref, w2_ref, hts2_ref):
    dinv = dinv_ref[...]
    pre = dinv * (agg_ref[0] + agg_ref[1] + hts_ref[...]) + b_ref[...]
    h1 = jnp.maximum(pre, 0.0)
    hts2_ref[...] = jnp.dot(h1, w2_ref[...], preferred_element_type=jnp.float32) * dinv


def _tc3_body(agg_ref, hts_ref, dinv_ref, b_ref, batch_ref, wfc_ref, bfc_ref, out_ref):
    dinv = dinv_ref[...]
    pre = dinv * (agg_ref[0] + agg_ref[1] + hts_ref[...]) + b_ref[...]
    h2 = jnp.maximum(pre, 0.0)
    gids = lax.broadcasted_iota(jnp.int32, (G, NP), 0)
    mask = (batch_ref[...] == gids).astype(jnp.float32)
    sums = jnp.dot(mask, h2, preferred_element_type=jnp.float32)
    counts = jnp.sum(mask, axis=1, keepdims=True)
    pooled = sums / jnp.maximum(counts, 1.0)
    out_ref[...] = jnp.dot(pooled, wfc_ref[...],
                           preferred_element_type=jnp.float32) + bfc_ref[...]


_tc1 = pl.pallas_call(
    _tc1_body,
    out_shape=(jax.ShapeDtypeStruct((NP, H), jnp.float32),
               jax.ShapeDtypeStruct((NP, 1), jnp.float32)),
)

_tc2 = pl.pallas_call(
    _tc2_body,
    out_shape=jax.ShapeDtypeStruct((NP, H), jnp.float32),
)

_tc3 = pl.pallas_call(
    _tc3_body,
    out_shape=jax.ShapeDtypeStruct((G, OUT), jnp.float32),
)


def kernel(x, edge_index, edge_attr, batch, node_ids, emb_table, W1, b1, W2, b2, Wfc, bfc):
    src3 = jnp.concatenate(
        [edge_index[0], jnp.zeros((EP - E,), jnp.int32)]).reshape(NW, CH, K)
    dst3 = jnp.concatenate(
        [edge_index[1], jnp.zeros((EP - E,), jnp.int32)]).reshape(NW, CH, K)
    w3 = jnp.concatenate(
        [edge_attr, jnp.zeros((EP - E,), jnp.float32)]).reshape(NW, CH, K)
    ids_p = jnp.concatenate([node_ids, jnp.zeros((NP - N,), jnp.int32)])
    x_p = jnp.concatenate([x, jnp.zeros((NP - N, D), jnp.float32)])
    batch_p = jnp.concatenate(
        [batch, jnp.full((NP - N,), G, jnp.int32)]).reshape(1, NP)
    W1a = W1[:D]
    W1b = W1[D:]
    b1r = b1.reshape(1, H)
    b2r = b2.reshape(1, H)
    bfcr = bfc.reshape(1, OUT)

    degp, embed = _sc_deg_embed(dst3, w3, ids_p, emb_table)
    degp3 = degp.reshape(NC, NP, 1)

    hts1, dinv = _tc1(x_p, embed, W1a, W1b, degp3)
    agg1 = _sc_edge_agg(src3, dst3, w3, hts1)
    hts2 = _tc2(agg1, hts1, dinv, b1r, W2)
    agg2 = _sc_edge_agg(src3, dst3, w3, hts2)
    return _tc3(agg2, hts2, dinv, b2r, batch_p, Wfc, bfcr)
